# baseline (speedup 1.0000x reference)
import functools

import jax
import jax.numpy as jnp
from jax import lax
from jax.experimental import pallas as pl
from jax.experimental.pallas import tpu as pltpu
from jax.experimental.pallas import tpu_sc as plsc

B = 1024
EMB = 32
NC = 2
NS = 16
NW = NC * NS
BPW = B // NW


def _sc_body(uid_hbm, iid_hbm, ue_hbm, ie_hbm, ub_hbm, ib_hbm, ut_hbm,
             y_hbm, t_hbm,
             uid_v, iid_v, ue_v, ie_v, ub_v, ib_v, ut_v, prod_v, y_v, sem):
    wid = lax.axis_index("s") * NC + lax.axis_index("c")
    base = wid * BPW

    pltpu.sync_copy(uid_hbm.at[pl.ds(base, BPW)], uid_v)
    pltpu.sync_copy(iid_hbm.at[pl.ds(base, BPW)], iid_v)

    c1 = pltpu.async_copy(ue_hbm.at[uid_v], ue_v, sem)
    c2 = pltpu.async_copy(ie_hbm.at[iid_v], ie_v, sem)
    c3 = pltpu.async_copy(ub_hbm.at[uid_v], ub_v, sem)
    c4 = pltpu.async_copy(ib_hbm.at[iid_v], ib_v, sem)
    c5 = pltpu.async_copy(ut_hbm.at[uid_v], ut_v, sem)
    c1.wait()
    c2.wait()
    c3.wait()
    c4.wait()
    c5.wait()

    for r in range(BPW):
        prod_v[pl.ds(r * 16, 16)] = (
            ue_v[r, pl.ds(0, 16)] * ie_v[r, pl.ds(0, 16)]
            + ue_v[r, pl.ds(16, 16)] * ie_v[r, pl.ds(16, 16)])

    lanes = jnp.arange(16, dtype=jnp.int32)
    for g in range(BPW // 16):
        rowbase = (g * 16 + lanes) * 16
        acc = ub_v[pl.ds(g * 16, 16)] + ib_v[pl.ds(g * 16, 16)]
        for l in range(16):
            acc = acc + plsc.load_gather(prod_v, [rowbase + l])
        y_v[pl.ds(g * 16, 16)] = acc

    pltpu.sync_copy(y_v, y_hbm.at[pl.ds(base, BPW)])
    pltpu.sync_copy(ut_v, t_hbm.at[pl.ds(base, BPW)])


@jax.jit
def _sc_stage(user_ids, item_ids, user_embeddings, item_embeddings,
              user_biases, item_biases, user_ts):
    mesh = plsc.VectorSubcoreMesh(core_axis_name="c", subcore_axis_name="s")
    return pl.kernel(
        _sc_body,
        out_type=(
            jax.ShapeDtypeStruct((B,), jnp.float32),
            jax.ShapeDtypeStruct((B,), jnp.float32),
        ),
        mesh=mesh,
        scratch_types=[
            pltpu.VMEM((BPW,), jnp.int32),
            pltpu.VMEM((BPW,), jnp.int32),
            pltpu.VMEM((BPW, EMB), jnp.float32),
            pltpu.VMEM((BPW, EMB), jnp.float32),
            pltpu.VMEM((BPW,), jnp.float32),
            pltpu.VMEM((BPW,), jnp.float32),
            pltpu.VMEM((BPW,), jnp.float32),
            pltpu.VMEM((BPW * 16,), jnp.float32),
            pltpu.VMEM((BPW,), jnp.float32),
            pltpu.SemaphoreType.DMA,
        ],
        compiler_params=pltpu.CompilerParams(needs_layout_passes=False,
                                             use_tc_tiling_on_sc=False),
    )(user_ids, item_ids, user_embeddings, item_embeddings,
      user_biases, item_biases, user_ts)


def _tc_body(y_ref, t_ref, out_ref):
    out_ref[...] = 1.0 / (1.0 + jnp.exp(y_ref[...] - t_ref[...]))


@jax.jit
def _tc_stage(y, t):
    rows = 128
    grid = B // rows
    return pl.pallas_call(
        _tc_body,
        grid=(grid,),
        in_specs=[
            pl.BlockSpec((1, B), lambda i: (0, 0)),
            pl.BlockSpec((rows, 1), lambda i: (i, 0)),
        ],
        out_specs=pl.BlockSpec((rows, B), lambda i: (i, 0)),
        out_shape=jax.ShapeDtypeStruct((B, B), jnp.float32),
    )(y.reshape(1, B), t)


def kernel(user_ids, item_ids, user_embeddings, item_embeddings,
           user_biases, item_biases, user_ts, user_betas):
    del user_betas
    y, t = _sc_stage(user_ids.astype(jnp.int32), item_ids.astype(jnp.int32),
                     user_embeddings, item_embeddings,
                     user_biases.reshape(-1), item_biases.reshape(-1),
                     user_ts.reshape(-1))
    return _tc_stage(y, t.reshape(B, 1))


# speedup vs baseline: 2.7543x; 2.7543x over previous
import jax
import jax.numpy as jnp
from jax import lax
from jax.experimental import pallas as pl
from jax.experimental.pallas import tpu as pltpu
from jax.experimental.pallas import tpu_sc as plsc

B = 1024
EMB = 32
LANES = 128
NC = 2
NS = 16
NW = NC * NS
BPW = B // NW
NG = BPW // 16
IPS = 4


def _sc_body(uid_hbm, iid_hbm, ub_hbm, ib_hbm, ut_hbm,
             b_hbm, t_hbm,
             uid_v, iid_v, ub_v, ib_v, ut_v, b_v, sem):
    wid = lax.axis_index("s") * NC + lax.axis_index("c")
    base = wid * BPW

    pltpu.sync_copy(uid_hbm.at[pl.ds(base, BPW)], uid_v)
    pltpu.sync_copy(iid_hbm.at[pl.ds(base, BPW)], iid_v)

    c1 = pltpu.async_copy(ub_hbm.at[uid_v], ub_v, sem)
    c2 = pltpu.async_copy(ib_hbm.at[iid_v], ib_v, sem)
    c3 = pltpu.async_copy(ut_hbm.at[uid_v], ut_v, sem)
    c1.wait()
    c2.wait()
    c3.wait()

    for g in range(NG):
        s = pl.ds(g * 16, 16)
        b_v[s] = ub_v[s] + ib_v[s]

    pltpu.sync_copy(b_v, b_hbm.at[pl.ds(base, BPW)])
    pltpu.sync_copy(ut_v, t_hbm.at[pl.ds(base, BPW)])


def _gather_body(uids_ref, iids_ref, *refs):
    urefs = refs[:IPS]
    irefs = refs[IPS:2 * IPS]
    y_ref = refs[2 * IPS]
    i = pl.program_id(0)
    iota = lax.broadcasted_iota(jnp.int32, (EMB, LANES), 1)
    for j in range(IPS):
        uid = uids_ref[i * IPS + j]
        iid = iids_ref[i * IPS + j]
        um = iota == uid % LANES
        im = iota == iid % LANES
        uv = jnp.sum(jnp.where(um, urefs[j][...], 0.0), axis=1)
        iv = jnp.sum(jnp.where(im, irefs[j][...], 0.0), axis=1)
        y_ref[i * IPS + j] = jnp.sum(uv * iv)


def _tc_body(y_ref, b_ref, t_ref, out_ref):
    out_ref[...] = 1.0 / (1.0 + jnp.exp(y_ref[...] + b_ref[...] - t_ref[...]))


@jax.jit
def _impl(user_ids, item_ids, user_embeddings, item_embeddings,
          user_biases, item_biases, user_ts):
    uids = user_ids.astype(jnp.int32)
    iids = item_ids.astype(jnp.int32)
    mesh = plsc.VectorSubcoreMesh(core_axis_name="c", subcore_axis_name="s")
    bsum, t = pl.kernel(
        _sc_body,
        out_type=(
            jax.ShapeDtypeStruct((B,), jnp.float32),
            jax.ShapeDtypeStruct((B,), jnp.float32),
        ),
        mesh=mesh,
        scratch_types=[
            pltpu.VMEM((BPW,), jnp.int32),
            pltpu.VMEM((BPW,), jnp.int32),
            pltpu.VMEM((BPW,), jnp.float32),
            pltpu.VMEM((BPW,), jnp.float32),
            pltpu.VMEM((BPW,), jnp.float32),
            pltpu.VMEM((BPW,), jnp.float32),
            pltpu.SemaphoreType.DMA,
        ],
        compiler_params=pltpu.CompilerParams(needs_layout_passes=False,
                                             use_tc_tiling_on_sc=False),
    )(uids, iids,
      user_biases.reshape(-1), item_biases.reshape(-1), user_ts.reshape(-1))

    ut = user_embeddings.T
    it = item_embeddings.T

    grid_spec = pltpu.PrefetchScalarGridSpec(
        num_scalar_prefetch=2,
        grid=(B // IPS,),
        in_specs=(
            [pl.BlockSpec(
                (EMB, LANES),
                (lambda i, u, v, j=j: (0, u[i * IPS + j] // LANES)))
             for j in range(IPS)]
            + [pl.BlockSpec(
                (EMB, LANES),
                (lambda i, u, v, j=j: (0, v[i * IPS + j] // LANES)))
               for j in range(IPS)]
        ),
        out_specs=pl.BlockSpec(memory_space=pltpu.SMEM),
    )
    y = pl.pallas_call(
        _gather_body,
        grid_spec=grid_spec,
        out_shape=jax.ShapeDtypeStruct((B,), jnp.float32),
    )(uids, iids, *([ut] * IPS), *([it] * IPS))

    rows = 128
    return pl.pallas_call(
        _tc_body,
        grid=(B // rows,),
        in_specs=[
            pl.BlockSpec((1, B), lambda i: (0, 0)),
            pl.BlockSpec((1, B), lambda i: (0, 0)),
            pl.BlockSpec((rows, 1), lambda i: (i, 0)),
        ],
        out_specs=pl.BlockSpec((rows, B), lambda i: (i, 0)),
        out_shape=jax.ShapeDtypeStruct((B, B), jnp.float32),
    )(y.reshape(1, B), bsum.reshape(1, B), t.reshape(B, 1))


def kernel(user_ids, item_ids, user_embeddings, item_embeddings,
           user_biases, item_biases, user_ts, user_betas):
    del user_betas
    return _impl(user_ids, item_ids, user_embeddings, item_embeddings,
                 user_biases, item_biases, user_ts)


# speedup vs baseline: 5.1662x; 1.8757x over previous
import jax
import jax.numpy as jnp
from jax import lax
from jax.experimental import pallas as pl
from jax.experimental.pallas import tpu as pltpu
from jax.experimental.pallas import tpu_sc as plsc

B = 1024
EMB = 32
LANES = 128
NC = 2
NS = 16
NW = NC * NS
BPW = B // NW
SB = 8
NSB = BPW // SB


def _extract(chunk16, lane):
    mask = lax.iota(jnp.int32, 16) == lane
    return jnp.max(jnp.where(mask, chunk16, jnp.int32(-2147483648)))


def _sc_body(uid_hbm, iid_hbm, uet_hbm, iet_hbm, ub_hbm, ib_hbm, ut_hbm,
             y_hbm, t_hbm,
             uid_v, iid_v, ustr_v, istr_v, prod_v, ub_v, ib_v, ut_v,
             y_v, sem, bsem):
    wid = lax.axis_index("s") * NC + lax.axis_index("c")
    base = wid * BPW

    pltpu.sync_copy(uid_hbm.at[pl.ds(base, BPW)], uid_v)
    pltpu.sync_copy(iid_hbm.at[pl.ds(base, BPW)], iid_v)

    b1 = pltpu.async_copy(ub_hbm.at[uid_v], ub_v, bsem)
    b2 = pltpu.async_copy(ib_hbm.at[iid_v], ib_v, bsem)
    b3 = pltpu.async_copy(ut_hbm.at[uid_v], ut_v, bsem)

    lanes = lax.iota(jnp.int32, 16)
    for sb in range(NSB):
        ucols = []
        icols = []
        copies = []
        uchunk = uid_v[pl.ds((sb * SB // 16) * 16, 16)]
        ichunk = iid_v[pl.ds((sb * SB // 16) * 16, 16)]
        for k in range(SB):
            lane = (sb * SB + k) % 16
            uidk = _extract(uchunk, lane)
            iidk = _extract(ichunk, lane)
            ucols.append(uidk % LANES)
            icols.append(iidk % LANES)
            ustripe = (uidk // LANES) * LANES
            istripe = (iidk // LANES) * LANES
            copies.append(pltpu.async_copy(
                uet_hbm.at[:, pl.ds(ustripe, LANES)], ustr_v.at[k], sem))
            copies.append(pltpu.async_copy(
                iet_hbm.at[:, pl.ds(istripe, LANES)], istr_v.at[k], sem))
        for c in copies:
            c.wait()
        for k in range(SB):
            k16 = jnp.zeros((16,), jnp.int32) + k
            cu = jnp.zeros((16,), jnp.int32) + ucols[k]
            ci = jnp.zeros((16,), jnp.int32) + icols[k]
            u_lo = plsc.load_gather(ustr_v, [k16, lanes, cu])
            u_hi = plsc.load_gather(ustr_v, [k16, lanes + 16, cu])
            i_lo = plsc.load_gather(istr_v, [k16, lanes, ci])
            i_hi = plsc.load_gather(istr_v, [k16, lanes + 16, ci])
            prod_v[pl.ds((sb * SB + k) * 16, 16)] = u_lo * i_lo + u_hi * i_hi

    b1.wait()
    b2.wait()
    b3.wait()

    for g in range(BPW // 16):
        s = pl.ds(g * 16, 16)
        rowbase = (g * 16 + lanes) * 16
        acc = ub_v[s] + ib_v[s]
        for l in range(16):
            acc = acc + plsc.load_gather(prod_v, [rowbase + l])
        y_v[s] = acc

    pltpu.sync_copy(y_v, y_hbm.at[pl.ds(base, BPW)])
    pltpu.sync_copy(ut_v, t_hbm.at[pl.ds(base, BPW)])


def _tc_body(y_ref, t_ref, out_ref):
    out_ref[...] = 1.0 / (1.0 + jnp.exp(y_ref[...] - t_ref[...]))


@jax.jit
def _impl(user_ids, item_ids, user_embeddings, item_embeddings,
          user_biases, item_biases, user_ts):
    uids = user_ids.astype(jnp.int32)
    iids = item_ids.astype(jnp.int32)
    mesh = plsc.VectorSubcoreMesh(core_axis_name="c", subcore_axis_name="s")
    y, t = pl.kernel(
        _sc_body,
        out_type=(
            jax.ShapeDtypeStruct((B,), jnp.float32),
            jax.ShapeDtypeStruct((B,), jnp.float32),
        ),
        mesh=mesh,
        scratch_types=[
            pltpu.VMEM((BPW,), jnp.int32),
            pltpu.VMEM((BPW,), jnp.int32),
            pltpu.VMEM((SB, EMB, LANES), jnp.float32),
            pltpu.VMEM((SB, EMB, LANES), jnp.float32),
            pltpu.VMEM((BPW * 16,), jnp.float32),
            pltpu.VMEM((BPW,), jnp.float32),
            pltpu.VMEM((BPW,), jnp.float32),
            pltpu.VMEM((BPW,), jnp.float32),
            pltpu.VMEM((BPW,), jnp.float32),
            pltpu.SemaphoreType.DMA,
            pltpu.SemaphoreType.DMA,
        ],
        compiler_params=pltpu.CompilerParams(needs_layout_passes=False,
                                             use_tc_tiling_on_sc=True),
    )(uids, iids,
      user_embeddings.T, item_embeddings.T,
      user_biases.reshape(-1), item_biases.reshape(-1), user_ts.reshape(-1))

    rows = 128
    return pl.pallas_call(
        _tc_body,
        grid=(B // rows,),
        in_specs=[
            pl.BlockSpec((1, B), lambda i: (0, 0)),
            pl.BlockSpec((rows, 1), lambda i: (i, 0)),
        ],
        out_specs=pl.BlockSpec((rows, B), lambda i: (i, 0)),
        out_shape=jax.ShapeDtypeStruct((B, B), jnp.float32),
    )(y.reshape(1, B), t.reshape(B, 1))


def kernel(user_ids, item_ids, user_embeddings, item_embeddings,
           user_biases, item_biases, user_ts, user_betas):
    del user_betas
    return _impl(user_ids, item_ids, user_embeddings, item_embeddings,
                 user_biases, item_biases, user_ts)


# speedup vs baseline: 19.0657x; 3.6905x over previous
import jax
import jax.numpy as jnp
from jax import lax
from jax.experimental import pallas as pl
from jax.experimental.pallas import tpu as pltpu
from jax.experimental.pallas import tpu_sc as plsc

B = 1024
EMB = 32
LANES = 128
NC = 2
NS = 16
NW = NC * NS
BPW = B // NW
SB = 4
NSB = BPW // SB
SBG = 16 // SB


def _extract(chunk16, lane):
    mask = lax.iota(jnp.int32, 16) == lane
    return jnp.max(jnp.where(mask, chunk16, jnp.int32(-2147483648)))


def _sc_body(uid_hbm, iid_hbm, uet_hbm, iet_hbm, ubt_hbm, ibt_hbm, utt_hbm,
             y_hbm, t_hbm,
             uid_v, iid_v, ucol_v, icol_v, ustr_v, istr_v,
             ubs_v, ibs_v, uts_v, prod_v, bsum_v, y_v, t_v, sem0, sem1):
    wid = lax.axis_index("s") * NC + lax.axis_index("c")
    base = wid * BPW
    sems = (sem0, sem1)

    pltpu.sync_copy(uid_hbm.at[pl.ds(base, BPW)], uid_v)
    pltpu.sync_copy(iid_hbm.at[pl.ds(base, BPW)], iid_v)

    lanes = lax.iota(jnp.int32, 16)
    zeros = jnp.zeros((16,), jnp.int32)
    for c in range(BPW // 16):
        s = pl.ds(c * 16, 16)
        ucol_v[s] = uid_v[s] % LANES
        icol_v[s] = iid_v[s] % LANES

    def fire(sb):
        g = sb // SBG
        q = sb % SBG
        buf = sb % 2
        sem = sems[buf]
        uchunk = uid_v[pl.ds(g * 16, 16)]
        ichunk = iid_v[pl.ds(g * 16, 16)]
        ucols = []
        icols = []
        copies = []
        for k in range(SB):
            lane = q * SB + k
            uidk = _extract(uchunk, lane)
            iidk = _extract(ichunk, lane)
            ucols.append(uidk % LANES)
            icols.append(iidk % LANES)
            ustripe = (uidk // LANES) * LANES
            istripe = (iidk // LANES) * LANES
            copies.append(pltpu.async_copy(
                uet_hbm.at[:, pl.ds(ustripe, LANES)], ustr_v.at[buf, k], sem))
            copies.append(pltpu.async_copy(
                iet_hbm.at[:, pl.ds(istripe, LANES)], istr_v.at[buf, k], sem))
            copies.append(pltpu.async_copy(
                ubt_hbm.at[:, pl.ds(ustripe, LANES)], ubs_v.at[lane], sem))
            copies.append(pltpu.async_copy(
                ibt_hbm.at[:, pl.ds(istripe, LANES)], ibs_v.at[lane], sem))
            copies.append(pltpu.async_copy(
                utt_hbm.at[:, pl.ds(ustripe, LANES)], uts_v.at[lane], sem))
        return (sb, copies, ucols, icols)

    def drain_extract(st):
        sb, copies, ucols, icols = st
        g = sb // SBG
        buf = sb % 2
        for cc in copies:
            cc.wait()
        for k in range(SB):
            bk16 = jnp.zeros((16,), jnp.int32) + buf
            k16 = jnp.zeros((16,), jnp.int32) + k
            cu = jnp.zeros((16,), jnp.int32) + ucols[k]
            ci = jnp.zeros((16,), jnp.int32) + icols[k]
            u_lo = plsc.load_gather(ustr_v, [bk16, k16, lanes, cu])
            u_hi = plsc.load_gather(ustr_v, [bk16, k16, lanes + 16, cu])
            i_lo = plsc.load_gather(istr_v, [bk16, k16, lanes, ci])
            i_hi = plsc.load_gather(istr_v, [bk16, k16, lanes + 16, ci])
            prod_v[pl.ds((sb * SB + k) * 16, 16)] = (
                u_lo * i_lo + u_hi * i_hi)
        if sb % SBG == SBG - 1:
            sg = pl.ds(g * 16, 16)
            cu16 = ucol_v[sg]
            ci16 = icol_v[sg]
            bsum_v[sg] = (plsc.load_gather(ubs_v, [lanes, zeros, cu16])
                          + plsc.load_gather(ibs_v, [lanes, zeros, ci16]))
            t_v[sg] = plsc.load_gather(uts_v, [lanes, zeros, cu16])

    prev = fire(0)
    for sb in range(1, NSB):
        cur = fire(sb)
        drain_extract(prev)
        prev = cur
    drain_extract(prev)

    for g in range(BPW // 16):
        s = pl.ds(g * 16, 16)
        rowbase = (g * 16 + lanes) * 16
        acc = bsum_v[s]
        for l in range(16):
            acc = acc + plsc.load_gather(prod_v, [rowbase + l])
        y_v[s] = acc

    pltpu.sync_copy(y_v, y_hbm.at[pl.ds(base, BPW)])
    pltpu.sync_copy(t_v, t_hbm.at[pl.ds(base, BPW)])


def _tc_body(y_ref, t_ref, out_ref):
    out_ref[...] = 1.0 / (1.0 + jnp.exp(y_ref[...] - t_ref[...]))


@jax.jit
def _impl(user_ids, item_ids, user_embeddings, item_embeddings,
          user_biases, item_biases, user_ts):
    uids = user_ids.astype(jnp.int32)
    iids = item_ids.astype(jnp.int32)
    mesh = plsc.VectorSubcoreMesh(core_axis_name="c", subcore_axis_name="s")
    y, t = pl.kernel(
        _sc_body,
        out_type=(
            jax.ShapeDtypeStruct((B,), jnp.float32),
            jax.ShapeDtypeStruct((B,), jnp.float32),
        ),
        mesh=mesh,
        scratch_types=[
            pltpu.VMEM((BPW,), jnp.int32),
            pltpu.VMEM((BPW,), jnp.int32),
            pltpu.VMEM((BPW,), jnp.int32),
            pltpu.VMEM((BPW,), jnp.int32),
            pltpu.VMEM((2, SB, EMB, LANES), jnp.float32),
            pltpu.VMEM((2, SB, EMB, LANES), jnp.float32),
            pltpu.VMEM((16, 1, LANES), jnp.float32),
            pltpu.VMEM((16, 1, LANES), jnp.float32),
            pltpu.VMEM((16, 1, LANES), jnp.float32),
            pltpu.VMEM((BPW * 16,), jnp.float32),
            pltpu.VMEM((BPW,), jnp.float32),
            pltpu.VMEM((BPW,), jnp.float32),
            pltpu.VMEM((BPW,), jnp.float32),
            pltpu.SemaphoreType.DMA,
            pltpu.SemaphoreType.DMA,
        ],
        compiler_params=pltpu.CompilerParams(needs_layout_passes=False,
                                             use_tc_tiling_on_sc=True),
    )(uids, iids,
      user_embeddings.T, item_embeddings.T,
      user_biases.T, item_biases.T, user_ts.T)

    rows = 128
    return pl.pallas_call(
        _tc_body,
        grid=(B // rows,),
        in_specs=[
            pl.BlockSpec((1, B), lambda i: (0, 0)),
            pl.BlockSpec((rows, 1), lambda i: (i, 0)),
        ],
        out_specs=pl.BlockSpec((rows, B), lambda i: (i, 0)),
        out_shape=jax.ShapeDtypeStruct((B, B), jnp.float32),
    )(y.reshape(1, B), t.reshape(B, 1))


def kernel(user_ids, item_ids, user_embeddings, item_embeddings,
           user_biases, item_biases, user_ts, user_betas):
    del user_betas
    return _impl(user_ids, item_ids, user_embeddings, item_embeddings,
                 user_biases, item_biases, user_ts)
